# Initial kernel scaffold; baseline (speedup 1.0000x reference)
import jax
import jax.numpy as jnp
from jax.experimental import pallas as pl


def kernel(coords, coord_mask, res_idx, padding_mask, confidence, params):
    raise NotImplementedError("write your pallas kernel here")


# speedup vs baseline: 4.7197x; 4.7197x over previous
import functools

import jax
import jax.numpy as jnp
import numpy as np
from jax import lax
from jax.experimental import pallas as pl
from jax.experimental.pallas import tpu as pltpu
from jax.experimental.pallas import tpu_sc as plsc

B, L = 8, 1024
NS, NV = 128, 16
ES, EV = 32, 1
K = 30
N = B * L
E = N * K
NBLK = 128
NBLOCKS = N // NBLK
EBLK = NBLK * K
TD = NS + 3 * NV

_SC_NW = 32
_GCH = 128
_GCW = E // _SC_NW // _GCH


def _mm(a, b):
    return lax.dot_general(a, b, (((1,), (0,)), ((), ())),
                           precision=lax.Precision.HIGHEST,
                           preferred_element_type=jnp.float32)


def _tile(x):
    return jnp.concatenate([x] * K, axis=0)


def _ln(x, g, b):
    mu = jnp.mean(x, axis=-1, keepdims=True)
    var = jnp.mean((x - mu) ** 2, axis=-1, keepdims=True)
    return (x - mu) / jnp.sqrt(var + 1e-5) * g + b


def _vscale(Vp):
    return 1.0 / jnp.sqrt(
        jnp.mean(Vp[0] ** 2 + Vp[1] ** 2 + Vp[2] ** 2, axis=-1, keepdims=True)
        + 1e-8)


def _embed_node_body(ns_ref, nv_ref, cr_ref, wh, ws, bs, wv, wsv, bsv, wc,
                     out_ref):
    Vh = [_mm(nv_ref[:, 3 * c:3 * c + 3], wh[...]) for c in range(3)]
    vn = jnp.sqrt(Vh[0] ** 2 + Vh[1] ** 2 + Vh[2] ** 2 + 1e-8)
    s = _mm(ns_ref[...], ws[0:6]) + _mm(vn, ws[6:22]) + bs[...]
    gate = jax.nn.sigmoid(_mm(s, wsv[...]) + bsv[...])
    Vo = [_mm(v, wv[...]) * gate for v in Vh]
    s = jax.nn.relu(s) + _mm(cr_ref[...], wc[...])
    out_ref[...] = jnp.concatenate([s] + Vo, axis=1)


def _embed_edge_body(es_ref, ev_ref, wh, ws, bs, wv, wsv, bsv,
                     so_ref, vo_ref):
    whv = wh[...]
    Vh = [ev_ref[:, c:c + 1] * whv for c in range(3)]
    vn = jnp.sqrt(Vh[0] ** 2 + Vh[1] ** 2 + Vh[2] ** 2 + 1e-8)
    s = _mm(es_ref[...], ws[0:ES]) + vn * ws[ES:ES + 1] + bs[...]
    gate = jax.nn.sigmoid(_mm(s, wsv[...]) + bsv[...])
    wvv = wv[...]
    so_ref[...] = jax.nn.relu(s)
    vo_ref[...] = jnp.concatenate([v * wvv * gate for v in Vh], axis=1)


def _layer_body(tab, gath, esf, evf,
                wh1, ws1, bs1, wv1, wsv1, bsv1,
                wh2, ws2, bs2, wv2, wsv2, bsv2,
                wh3, ws3, bs3, wv3, wsv3, bsv3,
                g1, b1,
                fh1, fws1, fbs1, fwv1, fwsv1, fbsv1,
                fh2, fws2, fbs2, fwv2, fwsv2, fbsv2,
                g2, b2,
                out_ref):
    sblk = tab[:, :NS]
    Vd = [tab[:, NS + c * NV:NS + (c + 1) * NV] for c in range(3)]
    ss = gath[:, :NS]
    Vs = [gath[:, NS + c * NV:NS + (c + 1) * NV] for c in range(3)]
    es = esf[...]

    Vh = [_tile(_mm(Vd[c], wh1[0:NV])) + _mm(Vs[c], wh1[NV:2 * NV])
          + evf[:, c:c + 1] * wh1[2 * NV:2 * NV + 1] for c in range(3)]
    vn = jnp.sqrt(Vh[0] ** 2 + Vh[1] ** 2 + Vh[2] ** 2 + 1e-8)
    s1 = (_tile(_mm(sblk, ws1[0:NS])) + _mm(ss, ws1[NS:2 * NS])
          + _mm(es, ws1[2 * NS:2 * NS + ES])
          + _mm(vn, ws1[2 * NS + ES:]) + bs1[...])
    gate = jax.nn.sigmoid(_mm(s1, wsv1[...]) + bsv1[...])
    V1 = [_mm(v, wv1[...]) * gate for v in Vh]
    s1 = jax.nn.relu(s1)

    Vh2 = [_mm(v, wh2[...]) for v in V1]
    vn2 = jnp.sqrt(Vh2[0] ** 2 + Vh2[1] ** 2 + Vh2[2] ** 2 + 1e-8)
    s2 = _mm(s1, ws2[0:NS]) + _mm(vn2, ws2[NS:]) + bs2[...]
    gate2 = jax.nn.sigmoid(_mm(s2, wsv2[...]) + bsv2[...])
    V2 = [_mm(v, wv2[...]) * gate2 for v in Vh2]
    s2 = jax.nn.relu(s2)

    Vh3 = [_mm(v, wh3[...]) for v in V2]
    vn3 = jnp.sqrt(Vh3[0] ** 2 + Vh3[1] ** 2 + Vh3[2] ** 2 + 1e-8)
    s3 = _mm(s2, ws3[0:NS]) + _mm(vn3, ws3[NS:]) + bs3[...]
    gate3 = jax.nn.sigmoid(_mm(s3, wsv3[...]) + bsv3[...])
    V3 = [_mm(v, wv3[...]) * gate3 for v in Vh3]

    inv_k = np.float32(1.0 / K)
    agg_s = jnp.sum(s3.reshape(K, NBLK, NS), axis=0) * inv_k
    aggV = [jnp.sum(v.reshape(K, NBLK, NV), axis=0) * inv_k for v in V3]

    sN = _ln(sblk + agg_s, g1[...], b1[...])
    Vsum = [Vd[c] + aggV[c] for c in range(3)]
    den = _vscale(Vsum)
    VN = [v * den for v in Vsum]

    fVh = [_mm(v, fh1[...]) for v in VN]
    fvn = jnp.sqrt(fVh[0] ** 2 + fVh[1] ** 2 + fVh[2] ** 2 + 1e-8)
    fs = _mm(sN, fws1[0:NS]) + _mm(fvn, fws1[NS:]) + fbs1[...]
    fgate = jax.nn.sigmoid(_mm(fs, fwsv1[...]) + fbsv1[...])
    fV1 = [_mm(v, fwv1[...]) * fgate for v in fVh]
    fs = jax.nn.relu(fs)

    fVh2 = [_mm(v, fh2[...]) for v in fV1]
    fvn2 = jnp.sqrt(fVh2[0] ** 2 + fVh2[1] ** 2 + fVh2[2] ** 2 + 1e-8)
    fs2 = _mm(fs, fws2[0:2 * NS]) + _mm(fvn2, fws2[2 * NS:]) + fbs2[...]
    fgate2 = jax.nn.sigmoid(_mm(fs2, fwsv2[...]) + fbsv2[...])
    fV2 = [_mm(v, fwv2[...]) * fgate2 for v in fVh2]

    so = _ln(sN + fs2, g2[...], b2[...])
    Vsum2 = [VN[c] + fV2[c] for c in range(3)]
    den2 = _vscale(Vsum2)
    out_ref[...] = jnp.concatenate([so] + [v * den2 for v in Vsum2], axis=1)


def _full_spec(a):
    nd = a.ndim
    return pl.BlockSpec(a.shape, lambda i, _n=nd: (0,) * _n)


def _embed_node_call(node_s, nv, conf_rbf, p, w_conf):
    weights = (p["wh"], p["ws"], p["bs"].reshape(1, -1), p["wv"],
               p["wsv"], p["bsv"].reshape(1, -1), w_conf)
    return pl.pallas_call(
        _embed_node_body,
        grid=(NBLOCKS,),
        in_specs=[pl.BlockSpec((NBLK, 6), lambda i: (i, 0)),
                  pl.BlockSpec((NBLK, 9), lambda i: (i, 0)),
                  pl.BlockSpec((NBLK, 16), lambda i: (i, 0))]
                 + [_full_spec(w) for w in weights],
        out_specs=pl.BlockSpec((NBLK, TD), lambda i: (i, 0)),
        out_shape=jax.ShapeDtypeStruct((N, TD), jnp.float32),
    )(node_s, nv, conf_rbf, *weights)


def _embed_edge_call(e_s, e_v, p):
    weights = (p["wh"], p["ws"], p["bs"].reshape(1, -1), p["wv"],
               p["wsv"], p["bsv"].reshape(1, -1))
    return pl.pallas_call(
        _embed_edge_body,
        grid=(NBLOCKS,),
        in_specs=[pl.BlockSpec((EBLK, ES), lambda i: (i, 0)),
                  pl.BlockSpec((EBLK, 3), lambda i: (i, 0))]
                 + [_full_spec(w) for w in weights],
        out_specs=[pl.BlockSpec((EBLK, ES), lambda i: (i, 0)),
                   pl.BlockSpec((EBLK, 3), lambda i: (i, 0))],
        out_shape=[jax.ShapeDtypeStruct((E, ES), jnp.float32),
                   jax.ShapeDtypeStruct((E, 3), jnp.float32)],
    )(e_s, e_v, *weights)


def _layer_weights(lp):
    out = []
    for p in lp["msg"]:
        out += [p["wh"], p["ws"], p["bs"].reshape(1, -1), p["wv"],
                p["wsv"], p["bsv"].reshape(1, -1)]
    out += [lp["ln1"]["g"].reshape(1, -1), lp["ln1"]["b"].reshape(1, -1)]
    for p in lp["ff"]:
        out += [p["wh"], p["ws"], p["bs"].reshape(1, -1), p["wv"],
                p["wsv"], p["bsv"].reshape(1, -1)]
    out += [lp["ln2"]["g"].reshape(1, -1), lp["ln2"]["b"].reshape(1, -1)]
    return out


def _layer_call(table, gath, esf, evf, weights):
    return pl.pallas_call(
        _layer_body,
        grid=(NBLOCKS,),
        in_specs=[pl.BlockSpec((NBLK, TD), lambda i: (i, 0)),
                  pl.BlockSpec((EBLK, TD), lambda i: (i, 0)),
                  pl.BlockSpec((EBLK, ES), lambda i: (i, 0)),
                  pl.BlockSpec((EBLK, 3), lambda i: (i, 0))]
                 + [_full_spec(w) for w in weights],
        out_specs=pl.BlockSpec((NBLK, TD), lambda i: (i, 0)),
        out_shape=jax.ShapeDtypeStruct((N, TD), jnp.float32),
    )(table, gath, esf, evf, *weights)


def _sc_gather(table, idx3d):
    mesh = plsc.VectorSubcoreMesh(core_axis_name="c", subcore_axis_name="s")

    @functools.partial(
        pl.kernel, mesh=mesh,
        out_type=jax.ShapeDtypeStruct((E, TD), jnp.float32),
        compiler_params=pltpu.CompilerParams(use_tc_tiling_on_sc=False),
        scratch_types=[pltpu.VMEM((_GCW, _GCH), jnp.int32),
                       pltpu.VMEM((_GCH, TD), jnp.float32),
                       pltpu.SemaphoreType.DMA])
    def gather_k(table_hbm, idx_hbm, out_hbm, idx_v, rows_v, sem):
        nc = mesh.num_cores
        wid = lax.axis_index("s") * nc + lax.axis_index("c")
        base = wid * (_GCW * _GCH)
        pltpu.sync_copy(idx_hbm.at[wid], idx_v)

        def step(j, carry):
            pltpu.async_copy(table_hbm.at[idx_v.at[j]], rows_v, sem).wait()
            pltpu.sync_copy(rows_v, out_hbm.at[pl.ds(base + j * _GCH, _GCH)])
            return carry

        lax.fori_loop(0, _GCW, step, 0)

    return gather_k(table, idx3d)


def _norm(x, axis=-1, keepdims=False, eps=1e-8):
    return jnp.sqrt(jnp.sum(x * x, axis=axis, keepdims=keepdims) + eps)


def _normalize(x, axis=-1):
    return x / _norm(x, axis=axis, keepdims=True)


def _rbf(d, d_min, d_max, d_count):
    mu = jnp.linspace(d_min, d_max, d_count)
    sigma = (d_max - d_min) / d_count
    return jnp.exp(-((d[..., None] - mu) / sigma) ** 2)


def _dihedrals(X):
    b = X.shape[0]
    Xf = X.reshape(b, -1, 3)
    dX = Xf[:, 1:] - Xf[:, :-1]
    U = _normalize(dX)
    u2, u1, u0 = U[:, :-2], U[:, 1:-1], U[:, 2:]
    n2 = _normalize(jnp.cross(u2, u1))
    n1 = _normalize(jnp.cross(u1, u0))
    cosD = jnp.clip(jnp.sum(n2 * n1, -1), -1 + 1e-7, 1 - 1e-7)
    D = jnp.sign(jnp.sum(u2 * n1, -1)) * jnp.arccos(cosD)
    D = jnp.pad(D, ((0, 0), (1, 2)))
    D = D.reshape(b, -1, 3)
    return jnp.concatenate([jnp.cos(D), jnp.sin(D)], -1)


def _orientations(Xca):
    f = _normalize(Xca[:, 1:] - Xca[:, :-1])
    bwd = _normalize(Xca[:, :-1] - Xca[:, 1:])
    f = jnp.pad(f, ((0, 0), (0, 1), (0, 0)))
    bwd = jnp.pad(bwd, ((0, 0), (1, 0), (0, 0)))
    return jnp.stack([f, bwd], axis=-2)


def _sidechains(X):
    n, ca, c = X[:, :, 0], X[:, :, 1], X[:, :, 2]
    c, n = _normalize(c - ca), _normalize(n - ca)
    bis = _normalize(c + n)
    perp = _normalize(jnp.cross(c, n))
    vec = -bis * (1.0 / np.sqrt(3.0)) - perp * np.sqrt(2.0 / 3.0)
    return vec[:, :, None, :]


def _knn_src(Xca, padding_mask, k):
    b, l = Xca.shape[0], Xca.shape[1]
    d = jnp.sum((Xca[:, :, None] - Xca[:, None, :]) ** 2, -1)
    m2 = padding_mask[:, None, :] | padding_mask[:, :, None]
    d = jnp.where(m2, 1e12, d)
    d = d + jnp.eye(l) * 1e12
    _, idx = jax.lax.top_k(-d, k)
    off = (jnp.arange(b) * l)[:, None, None]
    return (idx + off).reshape(b * l, k)


def _pos_embed(d, num=16):
    freq = jnp.exp(jnp.arange(0, num, 2, dtype=jnp.float32)
                   * (-np.log(10000.0) / num))
    ang = d[:, None].astype(jnp.float32) * freq
    return jnp.concatenate([jnp.cos(ang), jnp.sin(ang)], -1)


def kernel(coords, coord_mask, res_idx, padding_mask, confidence, params):
    b, l = coords.shape[0], coords.shape[1]
    Xca = coords[:, :, 1]
    node_s = _dihedrals(coords).reshape(N, 6)
    node_v = jnp.concatenate([_orientations(Xca), _sidechains(coords)],
                             axis=-2).reshape(N, 3, 3)
    cm = coord_mask.reshape(N).astype(jnp.float32)
    node_s = node_s * cm[:, None]
    node_v = node_v * cm[:, None, None]
    nv = jnp.concatenate([node_v[:, :, c] for c in range(3)], axis=1)
    conf_rbf = _rbf(confidence.reshape(N), 0.0, 1.0, 16)

    table = _embed_node_call(node_s, nv, conf_rbf, params["embed_node"],
                             params["w_conf"])

    src = _knn_src(Xca, padding_mask, K)
    src_perm = src.reshape(NBLOCKS, NBLK, K).transpose(0, 2, 1).reshape(E)
    dst_perm = jnp.broadcast_to(
        jnp.arange(N, dtype=src.dtype).reshape(NBLOCKS, 1, NBLK),
        (NBLOCKS, K, NBLK)).reshape(E)

    Xf = Xca.reshape(N, 3)
    evec = Xf[src_perm] - Xf[dst_perm]
    ri = res_idx.reshape(N)
    rd = ri[src_perm] - ri[dst_perm]
    e_s = jnp.concatenate([_rbf(_norm(evec), 0.0, 20.0, 16),
                           _pos_embed(rd, 16)], -1)
    e_v = _normalize(evec)
    esf, evf = _embed_edge_call(e_s, e_v, params["embed_edge"])

    idx3d = src_perm.astype(jnp.int32).reshape(_SC_NW, _GCW, _GCH)
    for lp in params["layers"]:
        gath = _sc_gather(table, idx3d)
        table = _layer_call(table, gath, esf, evf, _layer_weights(lp))

    s = table[:, :NS].reshape(b, l, NS)
    V = jnp.stack([table[:, NS + c * NV:NS + (c + 1) * NV]
                   for c in range(3)], axis=-1).reshape(b, l, NV, 3)
    return s, V


# speedup vs baseline: 6.5563x; 1.3891x over previous
import functools

import jax
import jax.numpy as jnp
import numpy as np
from jax import lax
from jax.experimental import pallas as pl
from jax.experimental.pallas import tpu as pltpu
from jax.experimental.pallas import tpu_sc as plsc

B, L = 8, 1024
NS, NV = 128, 16
ES, EV = 32, 1
K = 30
N = B * L
E = N * K
NBLK = 128
NBLOCKS = N // NBLK
EBLK = NBLK * K
TD = NS + 3 * NV

_SC_NW = 32
_GCH = 128
_GCW = E // _SC_NW // _GCH


def _mm(a, b):
    return lax.dot_general(a, b, (((1,), (0,)), ((), ())),
                           precision=lax.Precision.HIGHEST,
                           preferred_element_type=jnp.float32)


def _tile(x):
    return jnp.concatenate([x] * K, axis=0)


def _ln(x, g, b):
    mu = jnp.mean(x, axis=-1, keepdims=True)
    var = jnp.mean((x - mu) ** 2, axis=-1, keepdims=True)
    return (x - mu) / jnp.sqrt(var + 1e-5) * g + b


def _vscale(Vp):
    return 1.0 / jnp.sqrt(
        jnp.mean(Vp[0] ** 2 + Vp[1] ** 2 + Vp[2] ** 2, axis=-1, keepdims=True)
        + 1e-8)


def _embed_node_body(ns_ref, nv_ref, cr_ref, wh, ws, bs, wv, wsv, bsv, wc,
                     out_ref):
    Vh = [_mm(nv_ref[:, 3 * c:3 * c + 3], wh[...]) for c in range(3)]
    vn = jnp.sqrt(Vh[0] ** 2 + Vh[1] ** 2 + Vh[2] ** 2 + 1e-8)
    s = _mm(ns_ref[...], ws[0:6]) + _mm(vn, ws[6:22]) + bs[...]
    gate = jax.nn.sigmoid(_mm(s, wsv[...]) + bsv[...])
    Vo = [_mm(v, wv[...]) * gate for v in Vh]
    s = jax.nn.relu(s) + _mm(cr_ref[...], wc[...])
    out_ref[...] = jnp.concatenate([s] + Vo, axis=1)


def _embed_edge_body(es_ref, ev_ref, wh, ws, bs, wv, wsv, bsv,
                     so_ref, vo_ref):
    whv = wh[...]
    Vh = [ev_ref[:, c:c + 1] * whv for c in range(3)]
    vn = jnp.sqrt(Vh[0] ** 2 + Vh[1] ** 2 + Vh[2] ** 2 + 1e-8)
    s = _mm(es_ref[...], ws[0:ES]) + vn * ws[ES:ES + 1] + bs[...]
    gate = jax.nn.sigmoid(_mm(s, wsv[...]) + bsv[...])
    wvv = wv[...]
    so_ref[...] = jax.nn.relu(s)
    vo_ref[...] = jnp.concatenate([v * wvv * gate for v in Vh], axis=1)


def _layer_body(tab, gath, esf, evf,
                wh1, ws1, bs1, wv1, wsv1, bsv1,
                wh2, ws2, bs2, wv2, wsv2, bsv2,
                wh3, ws3, bs3, wv3, wsv3, bsv3,
                g1, b1,
                fh1, fws1, fbs1, fwv1, fwsv1, fbsv1,
                fh2, fws2, fbs2, fwv2, fwsv2, fbsv2,
                g2, b2,
                out_ref):
    sblk = tab[:, :NS]
    Vd = [tab[:, NS + c * NV:NS + (c + 1) * NV] for c in range(3)]
    ss = gath[:, :NS]
    Vs = [gath[:, NS + c * NV:NS + (c + 1) * NV] for c in range(3)]
    es = esf[...]

    Vh = [_tile(_mm(Vd[c], wh1[0:NV])) + _mm(Vs[c], wh1[NV:2 * NV])
          + evf[:, c:c + 1] * wh1[2 * NV:2 * NV + 1] for c in range(3)]
    vn = jnp.sqrt(Vh[0] ** 2 + Vh[1] ** 2 + Vh[2] ** 2 + 1e-8)
    s1 = (_tile(_mm(sblk, ws1[0:NS])) + _mm(ss, ws1[NS:2 * NS])
          + _mm(es, ws1[2 * NS:2 * NS + ES])
          + _mm(vn, ws1[2 * NS + ES:]) + bs1[...])
    gate = jax.nn.sigmoid(_mm(s1, wsv1[...]) + bsv1[...])
    V1 = [_mm(v, wv1[...]) * gate for v in Vh]
    s1 = jax.nn.relu(s1)

    Vh2 = [_mm(v, wh2[...]) for v in V1]
    vn2 = jnp.sqrt(Vh2[0] ** 2 + Vh2[1] ** 2 + Vh2[2] ** 2 + 1e-8)
    s2 = _mm(s1, ws2[0:NS]) + _mm(vn2, ws2[NS:]) + bs2[...]
    gate2 = jax.nn.sigmoid(_mm(s2, wsv2[...]) + bsv2[...])
    V2 = [_mm(v, wv2[...]) * gate2 for v in Vh2]
    s2 = jax.nn.relu(s2)

    Vh3 = [_mm(v, wh3[...]) for v in V2]
    vn3 = jnp.sqrt(Vh3[0] ** 2 + Vh3[1] ** 2 + Vh3[2] ** 2 + 1e-8)
    s3 = _mm(s2, ws3[0:NS]) + _mm(vn3, ws3[NS:]) + bs3[...]
    gate3 = jax.nn.sigmoid(_mm(s3, wsv3[...]) + bsv3[...])
    V3 = [_mm(v, wv3[...]) * gate3 for v in Vh3]

    inv_k = np.float32(1.0 / K)
    agg_s = jnp.sum(s3.reshape(K, NBLK, NS), axis=0) * inv_k
    aggV = [jnp.sum(v.reshape(K, NBLK, NV), axis=0) * inv_k for v in V3]

    sN = _ln(sblk + agg_s, g1[...], b1[...])
    Vsum = [Vd[c] + aggV[c] for c in range(3)]
    den = _vscale(Vsum)
    VN = [v * den for v in Vsum]

    fVh = [_mm(v, fh1[...]) for v in VN]
    fvn = jnp.sqrt(fVh[0] ** 2 + fVh[1] ** 2 + fVh[2] ** 2 + 1e-8)
    fs = _mm(sN, fws1[0:NS]) + _mm(fvn, fws1[NS:]) + fbs1[...]
    fgate = jax.nn.sigmoid(_mm(fs, fwsv1[...]) + fbsv1[...])
    fV1 = [_mm(v, fwv1[...]) * fgate for v in fVh]
    fs = jax.nn.relu(fs)

    fVh2 = [_mm(v, fh2[...]) for v in fV1]
    fvn2 = jnp.sqrt(fVh2[0] ** 2 + fVh2[1] ** 2 + fVh2[2] ** 2 + 1e-8)
    fs2 = _mm(fs, fws2[0:2 * NS]) + _mm(fvn2, fws2[2 * NS:]) + fbs2[...]
    fgate2 = jax.nn.sigmoid(_mm(fs2, fwsv2[...]) + fbsv2[...])
    fV2 = [_mm(v, fwv2[...]) * fgate2 for v in fVh2]

    so = _ln(sN + fs2, g2[...], b2[...])
    Vsum2 = [VN[c] + fV2[c] for c in range(3)]
    den2 = _vscale(Vsum2)
    out_ref[...] = jnp.concatenate([so] + [v * den2 for v in Vsum2], axis=1)


def _full_spec(a):
    nd = a.ndim
    return pl.BlockSpec(a.shape, lambda i, _n=nd: (0,) * _n)


def _embed_node_call(node_s, nv, conf_rbf, p, w_conf):
    weights = (p["wh"], p["ws"], p["bs"].reshape(1, -1), p["wv"],
               p["wsv"], p["bsv"].reshape(1, -1), w_conf)
    return pl.pallas_call(
        _embed_node_body,
        grid=(NBLOCKS,),
        in_specs=[pl.BlockSpec((NBLK, 6), lambda i: (i, 0)),
                  pl.BlockSpec((NBLK, 9), lambda i: (i, 0)),
                  pl.BlockSpec((NBLK, 16), lambda i: (i, 0))]
                 + [_full_spec(w) for w in weights],
        out_specs=pl.BlockSpec((NBLK, TD), lambda i: (i, 0)),
        out_shape=jax.ShapeDtypeStruct((N, TD), jnp.float32),
    )(node_s, nv, conf_rbf, *weights)


def _embed_edge_call(e_s, e_v, p):
    weights = (p["wh"], p["ws"], p["bs"].reshape(1, -1), p["wv"],
               p["wsv"], p["bsv"].reshape(1, -1))
    return pl.pallas_call(
        _embed_edge_body,
        grid=(NBLOCKS,),
        in_specs=[pl.BlockSpec((EBLK, ES), lambda i: (i, 0)),
                  pl.BlockSpec((EBLK, 3), lambda i: (i, 0))]
                 + [_full_spec(w) for w in weights],
        out_specs=[pl.BlockSpec((EBLK, ES), lambda i: (i, 0)),
                   pl.BlockSpec((EBLK, 3), lambda i: (i, 0))],
        out_shape=[jax.ShapeDtypeStruct((E, ES), jnp.float32),
                   jax.ShapeDtypeStruct((E, 3), jnp.float32)],
    )(e_s, e_v, *weights)


def _layer_weights(lp):
    out = []
    for p in lp["msg"]:
        out += [p["wh"], p["ws"], p["bs"].reshape(1, -1), p["wv"],
                p["wsv"], p["bsv"].reshape(1, -1)]
    out += [lp["ln1"]["g"].reshape(1, -1), lp["ln1"]["b"].reshape(1, -1)]
    for p in lp["ff"]:
        out += [p["wh"], p["ws"], p["bs"].reshape(1, -1), p["wv"],
                p["wsv"], p["bsv"].reshape(1, -1)]
    out += [lp["ln2"]["g"].reshape(1, -1), lp["ln2"]["b"].reshape(1, -1)]
    return out


def _layer_call(table, gath, esf, evf, weights):
    gw = gath.shape[1]
    return pl.pallas_call(
        _layer_body,
        grid=(NBLOCKS,),
        in_specs=[pl.BlockSpec((NBLK, TD), lambda i: (i, 0)),
                  pl.BlockSpec((EBLK, gw), lambda i: (i, 0)),
                  pl.BlockSpec((EBLK, ES), lambda i: (i, 0)),
                  pl.BlockSpec((EBLK, 3), lambda i: (i, 0))]
                 + [_full_spec(w) for w in weights],
        out_specs=pl.BlockSpec((NBLK, TD), lambda i: (i, 0)),
        out_shape=jax.ShapeDtypeStruct((N, TD), jnp.float32),
    )(table, gath, esf, evf, *weights)


def _sc_gather(table, idx3d):
    W = table.shape[1]
    mesh = plsc.VectorSubcoreMesh(core_axis_name="c", subcore_axis_name="s")

    @functools.partial(
        pl.kernel, mesh=mesh,
        out_type=jax.ShapeDtypeStruct((E, W), jnp.float32),
        compiler_params=pltpu.CompilerParams(use_tc_tiling_on_sc=False),
        scratch_types=[pltpu.VMEM((_GCW, _GCH), jnp.int32),
                       pltpu.VMEM((_GCH, W), jnp.float32),
                       pltpu.SemaphoreType.DMA])
    def gather_k(table_hbm, idx_hbm, out_hbm, idx_v, rows_v, sem):
        nc = mesh.num_cores
        wid = lax.axis_index("s") * nc + lax.axis_index("c")
        base = wid * (_GCW * _GCH)
        pltpu.sync_copy(idx_hbm.at[wid], idx_v)

        def step(j, carry):
            pltpu.async_copy(table_hbm.at[idx_v.at[j]], rows_v, sem).wait()
            pltpu.sync_copy(rows_v, out_hbm.at[pl.ds(base + j * _GCH, _GCH)])
            return carry

        lax.fori_loop(0, _GCW, step, 0)

    return gather_k(table, idx3d)


def _norm(x, axis=-1, keepdims=False, eps=1e-8):
    return jnp.sqrt(jnp.sum(x * x, axis=axis, keepdims=keepdims) + eps)


def _normalize(x, axis=-1):
    return x / _norm(x, axis=axis, keepdims=True)


def _rbf(d, d_min, d_max, d_count):
    mu = jnp.linspace(d_min, d_max, d_count)
    sigma = (d_max - d_min) / d_count
    return jnp.exp(-((d[..., None] - mu) / sigma) ** 2)


def _dihedrals(X):
    b = X.shape[0]
    Xf = X.reshape(b, -1, 3)
    dX = Xf[:, 1:] - Xf[:, :-1]
    U = _normalize(dX)
    u2, u1, u0 = U[:, :-2], U[:, 1:-1], U[:, 2:]
    n2 = _normalize(jnp.cross(u2, u1))
    n1 = _normalize(jnp.cross(u1, u0))
    cosD = jnp.clip(jnp.sum(n2 * n1, -1), -1 + 1e-7, 1 - 1e-7)
    D = jnp.sign(jnp.sum(u2 * n1, -1)) * jnp.arccos(cosD)
    D = jnp.pad(D, ((0, 0), (1, 2)))
    D = D.reshape(b, -1, 3)
    return jnp.concatenate([jnp.cos(D), jnp.sin(D)], -1)


def _orientations(Xca):
    f = _normalize(Xca[:, 1:] - Xca[:, :-1])
    bwd = _normalize(Xca[:, :-1] - Xca[:, 1:])
    f = jnp.pad(f, ((0, 0), (0, 1), (0, 0)))
    bwd = jnp.pad(bwd, ((0, 0), (1, 0), (0, 0)))
    return jnp.stack([f, bwd], axis=-2)


def _sidechains(X):
    n, ca, c = X[:, :, 0], X[:, :, 1], X[:, :, 2]
    c, n = _normalize(c - ca), _normalize(n - ca)
    bis = _normalize(c + n)
    perp = _normalize(jnp.cross(c, n))
    vec = -bis * (1.0 / np.sqrt(3.0)) - perp * np.sqrt(2.0 / 3.0)
    return vec[:, :, None, :]


def _knn_src(Xca, padding_mask, k):
    b, l = Xca.shape[0], Xca.shape[1]
    d = jnp.sum((Xca[:, :, None] - Xca[:, None, :]) ** 2, -1)
    m2 = padding_mask[:, None, :] | padding_mask[:, :, None]
    d = jnp.where(m2, 1e12, d)
    d = d + jnp.eye(l) * 1e12
    _, idx = jax.lax.top_k(-d, k)
    off = (jnp.arange(b) * l)[:, None, None]
    return (idx + off).reshape(b * l, k)


def _pos_embed(d, num=16):
    freq = jnp.exp(jnp.arange(0, num, 2, dtype=jnp.float32)
                   * (-np.log(10000.0) / num))
    ang = d[:, None].astype(jnp.float32) * freq
    return jnp.concatenate([jnp.cos(ang), jnp.sin(ang)], -1)


def kernel(coords, coord_mask, res_idx, padding_mask, confidence, params):
    b, l = coords.shape[0], coords.shape[1]
    Xca = coords[:, :, 1]
    node_s = _dihedrals(coords).reshape(N, 6)
    node_v = jnp.concatenate([_orientations(Xca), _sidechains(coords)],
                             axis=-2).reshape(N, 3, 3)
    cm = coord_mask.reshape(N).astype(jnp.float32)
    node_s = node_s * cm[:, None]
    node_v = node_v * cm[:, None, None]
    nv = jnp.concatenate([node_v[:, :, c] for c in range(3)], axis=1)
    conf_rbf = _rbf(confidence.reshape(N), 0.0, 1.0, 16)

    table = _embed_node_call(node_s, nv, conf_rbf, params["embed_node"],
                             params["w_conf"])

    src = _knn_src(Xca, padding_mask, K)
    src_perm = src.reshape(NBLOCKS, NBLK, K).transpose(0, 2, 1).reshape(E)
    idx3d = src_perm.astype(jnp.int32).reshape(_SC_NW, _GCW, _GCH)

    Xf = Xca.reshape(N, 3)
    rif = res_idx.reshape(N).astype(jnp.float32)
    geo = jnp.concatenate(
        [Xf, rif[:, None], jnp.zeros((N, 12), jnp.float32)], axis=1)
    gath0 = _sc_gather(jnp.concatenate([table, geo], axis=1), idx3d)

    evec = (gath0[:, TD:TD + 3]
            - jnp.broadcast_to(Xf.reshape(NBLOCKS, 1, NBLK, 3),
                               (NBLOCKS, K, NBLK, 3)).reshape(E, 3))
    rd = (gath0[:, TD + 3]
          - jnp.broadcast_to(rif.reshape(NBLOCKS, 1, NBLK),
                             (NBLOCKS, K, NBLK)).reshape(E))
    e_s = jnp.concatenate([_rbf(_norm(evec), 0.0, 20.0, 16),
                           _pos_embed(rd, 16)], -1)
    e_v = _normalize(evec)
    esf, evf = _embed_edge_call(e_s, e_v, params["embed_edge"])

    gath = gath0
    for lp in params["layers"]:
        table = _layer_call(table, gath, esf, evf, _layer_weights(lp))
        if lp is not params["layers"][-1]:
            gath = _sc_gather(table, idx3d)

    s = table[:, :NS].reshape(b, l, NS)
    V = jnp.stack([table[:, NS + c * NV:NS + (c + 1) * NV]
                   for c in range(3)], axis=-1).reshape(b, l, NV, 3)
    return s, V


# speedup vs baseline: 20.9998x; 3.2030x over previous
import functools

import jax
import jax.numpy as jnp
import numpy as np
from jax import lax
from jax.experimental import pallas as pl
from jax.experimental.pallas import tpu as pltpu
from jax.experimental.pallas import tpu_sc as plsc

B, L = 8, 1024
NS, NV = 128, 16
ES, EV = 32, 1
K = 30
N = B * L
E = N * K
NBLK = 128
NBLOCKS = N // NBLK
EBLK = NBLK * K
TD = NS + 3 * NV

_SC_NW = 32
_GCH = 128
_GCW = E // _SC_NW // _GCH


def _mm(a, b):
    return lax.dot_general(a, b, (((1,), (0,)), ((), ())),
                           precision=lax.Precision.DEFAULT,
                           preferred_element_type=jnp.float32)


def _tile(x):
    return jnp.concatenate([x] * K, axis=0)


def _ln(x, g, b):
    mu = jnp.mean(x, axis=-1, keepdims=True)
    var = jnp.mean((x - mu) ** 2, axis=-1, keepdims=True)
    return (x - mu) / jnp.sqrt(var + 1e-5) * g + b


def _vscale(Vp):
    return 1.0 / jnp.sqrt(
        jnp.mean(Vp[0] ** 2 + Vp[1] ** 2 + Vp[2] ** 2, axis=-1, keepdims=True)
        + 1e-8)


def _embed_node_body(ns_ref, nv_ref, cr_ref, wh, ws, bs, wv, wsv, bsv, wc,
                     out_ref):
    Vh = [_mm(nv_ref[:, 3 * c:3 * c + 3], wh[...]) for c in range(3)]
    vn = jnp.sqrt(Vh[0] ** 2 + Vh[1] ** 2 + Vh[2] ** 2 + 1e-8)
    s = _mm(ns_ref[...], ws[0:6]) + _mm(vn, ws[6:22]) + bs[...]
    gate = jax.nn.sigmoid(_mm(s, wsv[...]) + bsv[...])
    Vo = [_mm(v, wv[...]) * gate for v in Vh]
    s = jax.nn.relu(s) + _mm(cr_ref[...], wc[...])
    out_ref[...] = jnp.concatenate([s] + Vo, axis=1)


def _embed_edge_body(es_ref, ev_ref, wh, ws, bs, wv, wsv, bsv,
                     so_ref, vo_ref):
    whv = wh[...]
    Vh = [ev_ref[:, c:c + 1] * whv for c in range(3)]
    vn = jnp.sqrt(Vh[0] ** 2 + Vh[1] ** 2 + Vh[2] ** 2 + 1e-8)
    s = _mm(es_ref[...], ws[0:ES]) + vn * ws[ES:ES + 1] + bs[...]
    gate = jax.nn.sigmoid(_mm(s, wsv[...]) + bsv[...])
    wvv = wv[...]
    so_ref[...] = jax.nn.relu(s)
    vo_ref[...] = jnp.concatenate([v * wvv * gate for v in Vh], axis=1)


def _layer_body(tab, gath, esf, evf,
                wh1, ws1, bs1, wv1, wsv1, bsv1,
                wh2, ws2, bs2, wv2, wsv2, bsv2,
                wh3, ws3, bs3, wv3, wsv3, bsv3,
                g1, b1,
                fh1, fws1, fbs1, fwv1, fwsv1, fbsv1,
                fh2, fws2, fbs2, fwv2, fwsv2, fbsv2,
                g2, b2,
                out_ref):
    sblk = tab[:, :NS]
    Vd = [tab[:, NS + c * NV:NS + (c + 1) * NV] for c in range(3)]
    ss = gath[:, :NS]
    Vs = [gath[:, NS + c * NV:NS + (c + 1) * NV] for c in range(3)]
    es = esf[...]

    Vh = [_tile(_mm(Vd[c], wh1[0:NV])) + _mm(Vs[c], wh1[NV:2 * NV])
          + evf[:, c:c + 1] * wh1[2 * NV:2 * NV + 1] for c in range(3)]
    vn = jnp.sqrt(Vh[0] ** 2 + Vh[1] ** 2 + Vh[2] ** 2 + 1e-8)
    s1 = (_tile(_mm(sblk, ws1[0:NS])) + _mm(ss, ws1[NS:2 * NS])
          + _mm(es, ws1[2 * NS:2 * NS + ES])
          + _mm(vn, ws1[2 * NS + ES:]) + bs1[...])
    gate = jax.nn.sigmoid(_mm(s1, wsv1[...]) + bsv1[...])
    V1 = [_mm(v, wv1[...]) * gate for v in Vh]
    s1 = jax.nn.relu(s1)

    Vh2 = [_mm(v, wh2[...]) for v in V1]
    vn2 = jnp.sqrt(Vh2[0] ** 2 + Vh2[1] ** 2 + Vh2[2] ** 2 + 1e-8)
    s2 = _mm(s1, ws2[0:NS]) + _mm(vn2, ws2[NS:]) + bs2[...]
    gate2 = jax.nn.sigmoid(_mm(s2, wsv2[...]) + bsv2[...])
    V2 = [_mm(v, wv2[...]) * gate2 for v in Vh2]
    s2 = jax.nn.relu(s2)

    Vh3 = [_mm(v, wh3[...]) for v in V2]
    vn3 = jnp.sqrt(Vh3[0] ** 2 + Vh3[1] ** 2 + Vh3[2] ** 2 + 1e-8)
    s3 = _mm(s2, ws3[0:NS]) + _mm(vn3, ws3[NS:]) + bs3[...]
    gate3 = jax.nn.sigmoid(_mm(s3, wsv3[...]) + bsv3[...])
    V3 = [_mm(v, wv3[...]) * gate3 for v in Vh3]

    inv_k = np.float32(1.0 / K)
    agg_s = jnp.sum(s3.reshape(K, NBLK, NS), axis=0) * inv_k
    aggV = [jnp.sum(v.reshape(K, NBLK, NV), axis=0) * inv_k for v in V3]

    sN = _ln(sblk + agg_s, g1[...], b1[...])
    Vsum = [Vd[c] + aggV[c] for c in range(3)]
    den = _vscale(Vsum)
    VN = [v * den for v in Vsum]

    fVh = [_mm(v, fh1[...]) for v in VN]
    fvn = jnp.sqrt(fVh[0] ** 2 + fVh[1] ** 2 + fVh[2] ** 2 + 1e-8)
    fs = _mm(sN, fws1[0:NS]) + _mm(fvn, fws1[NS:]) + fbs1[...]
    fgate = jax.nn.sigmoid(_mm(fs, fwsv1[...]) + fbsv1[...])
    fV1 = [_mm(v, fwv1[...]) * fgate for v in fVh]
    fs = jax.nn.relu(fs)

    fVh2 = [_mm(v, fh2[...]) for v in fV1]
    fvn2 = jnp.sqrt(fVh2[0] ** 2 + fVh2[1] ** 2 + fVh2[2] ** 2 + 1e-8)
    fs2 = _mm(fs, fws2[0:2 * NS]) + _mm(fvn2, fws2[2 * NS:]) + fbs2[...]
    fgate2 = jax.nn.sigmoid(_mm(fs2, fwsv2[...]) + fbsv2[...])
    fV2 = [_mm(v, fwv2[...]) * fgate2 for v in fVh2]

    so = _ln(sN + fs2, g2[...], b2[...])
    Vsum2 = [VN[c] + fV2[c] for c in range(3)]
    den2 = _vscale(Vsum2)
    out_ref[...] = jnp.concatenate([so] + [v * den2 for v in Vsum2], axis=1)


def _full_spec(a):
    nd = a.ndim
    return pl.BlockSpec(a.shape, lambda i, _n=nd: (0,) * _n)


def _embed_node_call(node_s, nv, conf_rbf, p, w_conf):
    weights = (p["wh"], p["ws"], p["bs"].reshape(1, -1), p["wv"],
               p["wsv"], p["bsv"].reshape(1, -1), w_conf)
    return pl.pallas_call(
        _embed_node_body,
        grid=(NBLOCKS,),
        in_specs=[pl.BlockSpec((NBLK, 6), lambda i: (i, 0)),
                  pl.BlockSpec((NBLK, 9), lambda i: (i, 0)),
                  pl.BlockSpec((NBLK, 16), lambda i: (i, 0))]
                 + [_full_spec(w) for w in weights],
        out_specs=pl.BlockSpec((NBLK, TD), lambda i: (i, 0)),
        out_shape=jax.ShapeDtypeStruct((N, TD), jnp.float32),
    )(node_s, nv, conf_rbf, *weights)


def _embed_edge_call(e_s, e_v, p):
    weights = (p["wh"], p["ws"], p["bs"].reshape(1, -1), p["wv"],
               p["wsv"], p["bsv"].reshape(1, -1))
    return pl.pallas_call(
        _embed_edge_body,
        grid=(NBLOCKS,),
        in_specs=[pl.BlockSpec((EBLK, ES), lambda i: (i, 0)),
                  pl.BlockSpec((EBLK, 3), lambda i: (i, 0))]
                 + [_full_spec(w) for w in weights],
        out_specs=[pl.BlockSpec((EBLK, ES), lambda i: (i, 0)),
                   pl.BlockSpec((EBLK, 3), lambda i: (i, 0))],
        out_shape=[jax.ShapeDtypeStruct((E, ES), jnp.float32),
                   jax.ShapeDtypeStruct((E, 3), jnp.float32)],
    )(e_s, e_v, *weights)


def _layer_weights(lp):
    out = []
    for p in lp["msg"]:
        out += [p["wh"], p["ws"], p["bs"].reshape(1, -1), p["wv"],
                p["wsv"], p["bsv"].reshape(1, -1)]
    out += [lp["ln1"]["g"].reshape(1, -1), lp["ln1"]["b"].reshape(1, -1)]
    for p in lp["ff"]:
        out += [p["wh"], p["ws"], p["bs"].reshape(1, -1), p["wv"],
                p["wsv"], p["bsv"].reshape(1, -1)]
    out += [lp["ln2"]["g"].reshape(1, -1), lp["ln2"]["b"].reshape(1, -1)]
    return out


def _layer_call(table, gath, esf, evf, weights):
    gw = gath.shape[1]
    return pl.pallas_call(
        _layer_body,
        grid=(NBLOCKS,),
        in_specs=[pl.BlockSpec((NBLK, TD), lambda i: (i, 0)),
                  pl.BlockSpec((EBLK, gw), lambda i: (i, 0)),
                  pl.BlockSpec((EBLK, ES), lambda i: (i, 0)),
                  pl.BlockSpec((EBLK, 3), lambda i: (i, 0))]
                 + [_full_spec(w) for w in weights],
        out_specs=pl.BlockSpec((NBLK, TD), lambda i: (i, 0)),
        out_shape=jax.ShapeDtypeStruct((N, TD), jnp.float32),
    )(table, gath, esf, evf, *weights)


def _sc_gather(table, idx3d):
    W = table.shape[1]
    mesh = plsc.VectorSubcoreMesh(core_axis_name="c", subcore_axis_name="s")

    @functools.partial(
        pl.kernel, mesh=mesh,
        out_type=jax.ShapeDtypeStruct((E, W), jnp.float32),
        compiler_params=pltpu.CompilerParams(use_tc_tiling_on_sc=False),
        scratch_types=[pltpu.VMEM((_GCW, _GCH), jnp.int32),
                       pltpu.VMEM((_GCH, W), jnp.float32),
                       pltpu.SemaphoreType.DMA])
    def gather_k(table_hbm, idx_hbm, out_hbm, idx_v, rows_v, sem):
        nc = mesh.num_cores
        wid = lax.axis_index("s") * nc + lax.axis_index("c")
        base = wid * (_GCW * _GCH)
        pltpu.sync_copy(idx_hbm.at[wid], idx_v)

        def step(j, carry):
            pltpu.async_copy(table_hbm.at[idx_v.at[j]], rows_v, sem).wait()
            pltpu.sync_copy(rows_v, out_hbm.at[pl.ds(base + j * _GCH, _GCH)])
            return carry

        lax.fori_loop(0, _GCW, step, 0)

    return gather_k(table, idx3d)


def _norm(x, axis=-1, keepdims=False, eps=1e-8):
    return jnp.sqrt(jnp.sum(x * x, axis=axis, keepdims=keepdims) + eps)


def _normalize(x, axis=-1):
    return x / _norm(x, axis=axis, keepdims=True)


def _rbf(d, d_min, d_max, d_count):
    mu = jnp.linspace(d_min, d_max, d_count)
    sigma = (d_max - d_min) / d_count
    return jnp.exp(-((d[..., None] - mu) / sigma) ** 2)


def _dihedrals(X):
    b = X.shape[0]
    Xf = X.reshape(b, -1, 3)
    dX = Xf[:, 1:] - Xf[:, :-1]
    U = _normalize(dX)
    u2, u1, u0 = U[:, :-2], U[:, 1:-1], U[:, 2:]
    n2 = _normalize(jnp.cross(u2, u1))
    n1 = _normalize(jnp.cross(u1, u0))
    cosD = jnp.clip(jnp.sum(n2 * n1, -1), -1 + 1e-7, 1 - 1e-7)
    D = jnp.sign(jnp.sum(u2 * n1, -1)) * jnp.arccos(cosD)
    D = jnp.pad(D, ((0, 0), (1, 2)))
    D = D.reshape(b, -1, 3)
    return jnp.concatenate([jnp.cos(D), jnp.sin(D)], -1)


def _orientations(Xca):
    f = _normalize(Xca[:, 1:] - Xca[:, :-1])
    bwd = _normalize(Xca[:, :-1] - Xca[:, 1:])
    f = jnp.pad(f, ((0, 0), (0, 1), (0, 0)))
    bwd = jnp.pad(bwd, ((0, 0), (1, 0), (0, 0)))
    return jnp.stack([f, bwd], axis=-2)


def _sidechains(X):
    n, ca, c = X[:, :, 0], X[:, :, 1], X[:, :, 2]
    c, n = _normalize(c - ca), _normalize(n - ca)
    bis = _normalize(c + n)
    perp = _normalize(jnp.cross(c, n))
    vec = -bis * (1.0 / np.sqrt(3.0)) - perp * np.sqrt(2.0 / 3.0)
    return vec[:, :, None, :]


def _knn_src(Xca, padding_mask, k):
    b, l = Xca.shape[0], Xca.shape[1]
    d = jnp.sum((Xca[:, :, None] - Xca[:, None, :]) ** 2, -1)
    m2 = padding_mask[:, None, :] | padding_mask[:, :, None]
    d = jnp.where(m2, 1e12, d)
    d = d + jnp.eye(l) * 1e12
    _, idx = jax.lax.top_k(-d, k)
    off = (jnp.arange(b) * l)[:, None, None]
    return (idx + off).reshape(b * l, k)


def _pos_embed(d, num=16):
    freq = jnp.exp(jnp.arange(0, num, 2, dtype=jnp.float32)
                   * (-np.log(10000.0) / num))
    ang = d[:, None].astype(jnp.float32) * freq
    return jnp.concatenate([jnp.cos(ang), jnp.sin(ang)], -1)


def kernel(coords, coord_mask, res_idx, padding_mask, confidence, params):
    b, l = coords.shape[0], coords.shape[1]
    Xca = coords[:, :, 1]
    node_s = _dihedrals(coords).reshape(N, 6)
    node_v = jnp.concatenate([_orientations(Xca), _sidechains(coords)],
                             axis=-2).reshape(N, 3, 3)
    cm = coord_mask.reshape(N).astype(jnp.float32)
    node_s = node_s * cm[:, None]
    node_v = node_v * cm[:, None, None]
    nv = jnp.concatenate([node_v[:, :, c] for c in range(3)], axis=1)
    conf_rbf = _rbf(confidence.reshape(N), 0.0, 1.0, 16)

    table = _embed_node_call(node_s, nv, conf_rbf, params["embed_node"],
                             params["w_conf"])

    src = _knn_src(Xca, padding_mask, K)
    src_perm = src.reshape(NBLOCKS, NBLK, K).transpose(0, 2, 1).reshape(E)
    idx3d = src_perm.astype(jnp.int32).reshape(_SC_NW, _GCW, _GCH)

    Xf = Xca.reshape(N, 3)
    rif = res_idx.reshape(N).astype(jnp.float32)
    geo = jnp.concatenate(
        [Xf, rif[:, None], jnp.zeros((N, 12), jnp.float32)], axis=1)
    gath0 = _sc_gather(jnp.concatenate([table, geo], axis=1), idx3d)

    evec = (gath0[:, TD:TD + 3]
            - jnp.broadcast_to(Xf.reshape(NBLOCKS, 1, NBLK, 3),
                               (NBLOCKS, K, NBLK, 3)).reshape(E, 3))
    rd = (gath0[:, TD + 3]
          - jnp.broadcast_to(rif.reshape(NBLOCKS, 1, NBLK),
                             (NBLOCKS, K, NBLK)).reshape(E))
    e_s = jnp.concatenate([_rbf(_norm(evec), 0.0, 20.0, 16),
                           _pos_embed(rd, 16)], -1)
    e_v = _normalize(evec)
    esf, evf = _embed_edge_call(e_s, e_v, params["embed_edge"])

    gath = gath0
    for lp in params["layers"]:
        table = _layer_call(table, gath, esf, evf, _layer_weights(lp))
        if lp is not params["layers"][-1]:
            gath = _sc_gather(table, idx3d)

    s = table[:, :NS].reshape(b, l, NS)
    V = jnp.stack([table[:, NS + c * NV:NS + (c + 1) * NV]
                   for c in range(3)], axis=-1).reshape(b, l, NV, 3)
    return s, V


# speedup vs baseline: 22.9942x; 1.0950x over previous
import functools

import jax
import jax.numpy as jnp
import numpy as np
from jax import lax
from jax.experimental import pallas as pl
from jax.experimental.pallas import tpu as pltpu
from jax.experimental.pallas import tpu_sc as plsc

B, L = 8, 1024
NS, NV = 128, 16
ES, EV = 32, 1
K = 30
N = B * L
E = N * K
NBLK = 128
NBLOCKS = N // NBLK
EBLK = NBLK * K
TD = NS + 3 * NV

_SC_NW = 32
_GCH = 128
_GCW = E // _SC_NW // _GCH


def _mm(a, b):
    return lax.dot_general(a, b, (((1,), (0,)), ((), ())),
                           precision=lax.Precision.DEFAULT,
                           preferred_element_type=jnp.float32)


def _tile(x):
    return jnp.concatenate([x] * K, axis=0)


def _ln(x, g, b):
    mu = jnp.mean(x, axis=-1, keepdims=True)
    var = jnp.mean((x - mu) ** 2, axis=-1, keepdims=True)
    return (x - mu) / jnp.sqrt(var + 1e-5) * g + b


def _vscale(Vp):
    return 1.0 / jnp.sqrt(
        jnp.mean(Vp[0] ** 2 + Vp[1] ** 2 + Vp[2] ** 2, axis=-1, keepdims=True)
        + 1e-8)


def _embed_node_body(ns_ref, nv_ref, cr_ref, wh, ws, bs, wv, wsv, bsv, wc,
                     out_ref):
    Vh = [_mm(nv_ref[:, 3 * c:3 * c + 3], wh[...]) for c in range(3)]
    vn = jnp.sqrt(Vh[0] ** 2 + Vh[1] ** 2 + Vh[2] ** 2 + 1e-8)
    s = _mm(ns_ref[...], ws[0:6]) + _mm(vn, ws[6:22]) + bs[...]
    gate = jax.nn.sigmoid(_mm(s, wsv[...]) + bsv[...])
    Vo = [_mm(v, wv[...]) * gate for v in Vh]
    s = jax.nn.relu(s) + _mm(cr_ref[...], wc[...])
    out_ref[...] = jnp.concatenate([s] + Vo, axis=1)


def _embed_edge_body(es_ref, ev_ref, wh, ws, bs, wv, wsv, bsv,
                     so_ref, vo_ref):
    whv = wh[...]
    Vh = [ev_ref[:, c:c + 1] * whv for c in range(3)]
    vn = jnp.sqrt(Vh[0] ** 2 + Vh[1] ** 2 + Vh[2] ** 2 + 1e-8)
    s = _mm(es_ref[...], ws[0:ES]) + vn * ws[ES:ES + 1] + bs[...]
    gate = jax.nn.sigmoid(_mm(s, wsv[...]) + bsv[...])
    wvv = wv[...]
    so_ref[...] = jax.nn.relu(s)
    vo_ref[...] = jnp.concatenate([v * wvv * gate for v in Vh], axis=1)


_ST = 64


def _layer_body(tab, gath, esf, evf,
                whd1, whs1, whe1, wsd1, wss1, wse1, wsn1, bs1, wv1p, wsv1, bsv1,
                wh2p, wss2, wsn2, bs2, wv2p, wsv2, bsv2,
                wh3p, wss3, wsn3, bs3, wv3p, wsv3, bsv3,
                g1, b1,
                fh1, fws1, fbs1, fwv1, fwsv1, fbsv1,
                fh2, fws2, fbs2, fwv2, fwsv2, fbsv2,
                g2, b2,
                out_ref):
    sblk = tab[:, :NS]
    Vd48 = tab[:, NS:NS + 3 * NV]
    ss = gath[:, :NS]
    Vs48 = gath[:, NS:NS + 3 * NV]
    es = esf[...]
    ev = evf[...]

    Vh = (_tile(_mm(Vd48, whd1[...])) + _mm(Vs48, whs1[...])
          + _mm(ev, whe1[...]))
    h1 = [Vh[:, c * _ST:c * _ST + 33] for c in range(3)]
    vn = jnp.sqrt(h1[0] ** 2 + h1[1] ** 2 + h1[2] ** 2 + 1e-8)
    s1 = (_tile(_mm(sblk, wsd1[...])) + _mm(ss, wss1[...])
          + _mm(es, wse1[...]) + _mm(vn, wsn1[...]) + bs1[...])
    gate = jax.nn.sigmoid(_mm(s1, wsv1[...]) + bsv1[...])
    V1 = _mm(Vh, wv1p[...]) * jnp.concatenate([gate] * 3, axis=1)
    s1 = jax.nn.relu(s1)

    Vh2 = _mm(V1, wh2p[...])
    p2 = [Vh2[:, c * NV:(c + 1) * NV] for c in range(3)]
    vn2 = jnp.sqrt(p2[0] ** 2 + p2[1] ** 2 + p2[2] ** 2 + 1e-8)
    s2 = _mm(s1, wss2[...]) + _mm(vn2, wsn2[...]) + bs2[...]
    gate2 = jax.nn.sigmoid(_mm(s2, wsv2[...]) + bsv2[...])
    V2 = _mm(Vh2, wv2p[...]) * jnp.concatenate([gate2] * 3, axis=1)
    s2 = jax.nn.relu(s2)

    Vh3 = _mm(V2, wh3p[...])
    p3 = [Vh3[:, c * NV:(c + 1) * NV] for c in range(3)]
    vn3 = jnp.sqrt(p3[0] ** 2 + p3[1] ** 2 + p3[2] ** 2 + 1e-8)
    s3 = _mm(s2, wss3[...]) + _mm(vn3, wsn3[...]) + bs3[...]
    gate3 = jax.nn.sigmoid(_mm(s3, wsv3[...]) + bsv3[...])
    V3 = _mm(Vh3, wv3p[...]) * jnp.concatenate([gate3] * 3, axis=1)

    inv_k = np.float32(1.0 / K)
    agg_s = jnp.sum(s3.reshape(K, NBLK, NS), axis=0) * inv_k
    aggV48 = jnp.sum(V3.reshape(K, NBLK, 3 * NV), axis=0) * inv_k

    sN = _ln(sblk + agg_s, g1[...], b1[...])
    Vsum48 = Vd48 + aggV48
    q = [Vsum48[:, c * NV:(c + 1) * NV] for c in range(3)]
    den = _vscale(q)
    Vsum = q
    VN = [v * den for v in Vsum]

    fVh = [_mm(v, fh1[...]) for v in VN]
    fvn = jnp.sqrt(fVh[0] ** 2 + fVh[1] ** 2 + fVh[2] ** 2 + 1e-8)
    fs = _mm(sN, fws1[0:NS]) + _mm(fvn, fws1[NS:]) + fbs1[...]
    fgate = jax.nn.sigmoid(_mm(fs, fwsv1[...]) + fbsv1[...])
    fV1 = [_mm(v, fwv1[...]) * fgate for v in fVh]
    fs = jax.nn.relu(fs)

    fVh2 = [_mm(v, fh2[...]) for v in fV1]
    fvn2 = jnp.sqrt(fVh2[0] ** 2 + fVh2[1] ** 2 + fVh2[2] ** 2 + 1e-8)
    fs2 = _mm(fs, fws2[0:2 * NS]) + _mm(fvn2, fws2[2 * NS:]) + fbs2[...]
    fgate2 = jax.nn.sigmoid(_mm(fs2, fwsv2[...]) + fbsv2[...])
    fV2 = [_mm(v, fwv2[...]) * fgate2 for v in fVh2]

    so = _ln(sN + fs2, g2[...], b2[...])
    Vsum2 = [VN[c] + fV2[c] for c in range(3)]
    den2 = _vscale(Vsum2)
    out_ref[...] = jnp.concatenate([so] + [v * den2 for v in Vsum2], axis=1)


def _full_spec(a):
    nd = a.ndim
    return pl.BlockSpec(a.shape, lambda i, _n=nd: (0,) * _n)


def _embed_node_call(node_s, nv, conf_rbf, p, w_conf):
    weights = (p["wh"], p["ws"], p["bs"].reshape(1, -1), p["wv"],
               p["wsv"], p["bsv"].reshape(1, -1), w_conf)
    return pl.pallas_call(
        _embed_node_body,
        grid=(NBLOCKS,),
        in_specs=[pl.BlockSpec((NBLK, 6), lambda i: (i, 0)),
                  pl.BlockSpec((NBLK, 9), lambda i: (i, 0)),
                  pl.BlockSpec((NBLK, 16), lambda i: (i, 0))]
                 + [_full_spec(w) for w in weights],
        out_specs=pl.BlockSpec((NBLK, TD), lambda i: (i, 0)),
        out_shape=jax.ShapeDtypeStruct((N, TD), jnp.float32),
    )(node_s, nv, conf_rbf, *weights)


def _embed_edge_call(e_s, e_v, p):
    weights = (p["wh"], p["ws"], p["bs"].reshape(1, -1), p["wv"],
               p["wsv"], p["bsv"].reshape(1, -1))
    return pl.pallas_call(
        _embed_edge_body,
        grid=(NBLOCKS,),
        in_specs=[pl.BlockSpec((EBLK, ES), lambda i: (i, 0)),
                  pl.BlockSpec((EBLK, 3), lambda i: (i, 0))]
                 + [_full_spec(w) for w in weights],
        out_specs=[pl.BlockSpec((EBLK, ES), lambda i: (i, 0)),
                   pl.BlockSpec((EBLK, 3), lambda i: (i, 0))],
        out_shape=[jax.ShapeDtypeStruct((E, ES), jnp.float32),
                   jax.ShapeDtypeStruct((E, 3), jnp.float32)],
    )(e_s, e_v, *weights)


def _blockdiag(w, sw_in, sw_out):
    ki, ko = w.shape
    out = jnp.zeros((3 * sw_in, 3 * sw_out), w.dtype)
    for c in range(3):
        out = out.at[c * sw_in:c * sw_in + ki,
                     c * sw_out:c * sw_out + ko].set(w)
    return out


def _layer_weights(lp):
    p1, p2, p3 = lp["msg"]
    out = [_blockdiag(p1["wh"][0:NV], NV, _ST),
           _blockdiag(p1["wh"][NV:2 * NV], NV, _ST),
           _blockdiag(p1["wh"][2 * NV:2 * NV + 1], 1, _ST),
           p1["ws"][0:NS], p1["ws"][NS:2 * NS],
           p1["ws"][2 * NS:2 * NS + ES], p1["ws"][2 * NS + ES:],
           p1["bs"].reshape(1, -1),
           _blockdiag(p1["wv"], _ST, NV),
           p1["wsv"], p1["bsv"].reshape(1, -1)]
    for p in (p2, p3):
        out += [_blockdiag(p["wh"], NV, NV),
                p["ws"][0:NS], p["ws"][NS:],
                p["bs"].reshape(1, -1),
                _blockdiag(p["wv"], NV, NV),
                p["wsv"], p["bsv"].reshape(1, -1)]
    out += [lp["ln1"]["g"].reshape(1, -1), lp["ln1"]["b"].reshape(1, -1)]
    for p in lp["ff"]:
        out += [p["wh"], p["ws"], p["bs"].reshape(1, -1), p["wv"],
                p["wsv"], p["bsv"].reshape(1, -1)]
    out += [lp["ln2"]["g"].reshape(1, -1), lp["ln2"]["b"].reshape(1, -1)]
    return out


def _layer_call(table, gath, esf, evf, weights):
    gw = gath.shape[1]
    return pl.pallas_call(
        _layer_body,
        grid=(NBLOCKS,),
        in_specs=[pl.BlockSpec((NBLK, TD), lambda i: (i, 0)),
                  pl.BlockSpec((EBLK, gw), lambda i: (i, 0)),
                  pl.BlockSpec((EBLK, ES), lambda i: (i, 0)),
                  pl.BlockSpec((EBLK, 3), lambda i: (i, 0))]
                 + [_full_spec(w) for w in weights],
        out_specs=pl.BlockSpec((NBLK, TD), lambda i: (i, 0)),
        out_shape=jax.ShapeDtypeStruct((N, TD), jnp.float32),
    )(table, gath, esf, evf, *weights)


def _sc_gather(table, idx3d):
    W = table.shape[1]
    mesh = plsc.VectorSubcoreMesh(core_axis_name="c", subcore_axis_name="s")

    @functools.partial(
        pl.kernel, mesh=mesh,
        out_type=jax.ShapeDtypeStruct((E, W), jnp.float32),
        compiler_params=pltpu.CompilerParams(use_tc_tiling_on_sc=False),
        scratch_types=[pltpu.VMEM((_GCW, _GCH), jnp.int32),
                       pltpu.VMEM((_GCH, W), jnp.float32),
                       pltpu.SemaphoreType.DMA])
    def gather_k(table_hbm, idx_hbm, out_hbm, idx_v, rows_v, sem):
        nc = mesh.num_cores
        wid = lax.axis_index("s") * nc + lax.axis_index("c")
        base = wid * (_GCW * _GCH)
        pltpu.sync_copy(idx_hbm.at[wid], idx_v)

        def step(j, carry):
            pltpu.async_copy(table_hbm.at[idx_v.at[j]], rows_v, sem).wait()
            pltpu.sync_copy(rows_v, out_hbm.at[pl.ds(base + j * _GCH, _GCH)])
            return carry

        lax.fori_loop(0, _GCW, step, 0)

    return gather_k(table, idx3d)


def _norm(x, axis=-1, keepdims=False, eps=1e-8):
    return jnp.sqrt(jnp.sum(x * x, axis=axis, keepdims=keepdims) + eps)


def _normalize(x, axis=-1):
    return x / _norm(x, axis=axis, keepdims=True)


def _rbf(d, d_min, d_max, d_count):
    mu = jnp.linspace(d_min, d_max, d_count)
    sigma = (d_max - d_min) / d_count
    return jnp.exp(-((d[..., None] - mu) / sigma) ** 2)


def _dihedrals(X):
    b = X.shape[0]
    Xf = X.reshape(b, -1, 3)
    dX = Xf[:, 1:] - Xf[:, :-1]
    U = _normalize(dX)
    u2, u1, u0 = U[:, :-2], U[:, 1:-1], U[:, 2:]
    n2 = _normalize(jnp.cross(u2, u1))
    n1 = _normalize(jnp.cross(u1, u0))
    cosD = jnp.clip(jnp.sum(n2 * n1, -1), -1 + 1e-7, 1 - 1e-7)
    D = jnp.sign(jnp.sum(u2 * n1, -1)) * jnp.arccos(cosD)
    D = jnp.pad(D, ((0, 0), (1, 2)))
    D = D.reshape(b, -1, 3)
    return jnp.concatenate([jnp.cos(D), jnp.sin(D)], -1)


def _orientations(Xca):
    f = _normalize(Xca[:, 1:] - Xca[:, :-1])
    bwd = _normalize(Xca[:, :-1] - Xca[:, 1:])
    f = jnp.pad(f, ((0, 0), (0, 1), (0, 0)))
    bwd = jnp.pad(bwd, ((0, 0), (1, 0), (0, 0)))
    return jnp.stack([f, bwd], axis=-2)


def _sidechains(X):
    n, ca, c = X[:, :, 0], X[:, :, 1], X[:, :, 2]
    c, n = _normalize(c - ca), _normalize(n - ca)
    bis = _normalize(c + n)
    perp = _normalize(jnp.cross(c, n))
    vec = -bis * (1.0 / np.sqrt(3.0)) - perp * np.sqrt(2.0 / 3.0)
    return vec[:, :, None, :]


def _knn_src(Xca, padding_mask, k):
    b, l = Xca.shape[0], Xca.shape[1]
    d = jnp.sum((Xca[:, :, None] - Xca[:, None, :]) ** 2, -1)
    m2 = padding_mask[:, None, :] | padding_mask[:, :, None]
    d = jnp.where(m2, 1e12, d)
    d = d + jnp.eye(l) * 1e12
    _, idx = jax.lax.top_k(-d, k)
    off = (jnp.arange(b) * l)[:, None, None]
    return (idx + off).reshape(b * l, k)


def _pos_embed(d, num=16):
    freq = jnp.exp(jnp.arange(0, num, 2, dtype=jnp.float32)
                   * (-np.log(10000.0) / num))
    ang = d[:, None].astype(jnp.float32) * freq
    return jnp.concatenate([jnp.cos(ang), jnp.sin(ang)], -1)


def kernel(coords, coord_mask, res_idx, padding_mask, confidence, params):
    b, l = coords.shape[0], coords.shape[1]
    Xca = coords[:, :, 1]
    node_s = _dihedrals(coords).reshape(N, 6)
    node_v = jnp.concatenate([_orientations(Xca), _sidechains(coords)],
                             axis=-2).reshape(N, 3, 3)
    cm = coord_mask.reshape(N).astype(jnp.float32)
    node_s = node_s * cm[:, None]
    node_v = node_v * cm[:, None, None]
    nv = jnp.concatenate([node_v[:, :, c] for c in range(3)], axis=1)
    conf_rbf = _rbf(confidence.reshape(N), 0.0, 1.0, 16)

    table = _embed_node_call(node_s, nv, conf_rbf, params["embed_node"],
                             params["w_conf"])

    src = _knn_src(Xca, padding_mask, K)
    src_perm = src.reshape(NBLOCKS, NBLK, K).transpose(0, 2, 1).reshape(E)
    idx3d = src_perm.astype(jnp.int32).reshape(_SC_NW, _GCW, _GCH)

    Xf = Xca.reshape(N, 3)
    rif = res_idx.reshape(N).astype(jnp.float32)
    geo = jnp.concatenate(
        [Xf, rif[:, None], jnp.zeros((N, 12), jnp.float32)], axis=1)
    gath0 = _sc_gather(jnp.concatenate([table, geo], axis=1), idx3d)

    evec = (gath0[:, TD:TD + 3]
            - jnp.broadcast_to(Xf.reshape(NBLOCKS, 1, NBLK, 3),
                               (NBLOCKS, K, NBLK, 3)).reshape(E, 3))
    rd = (gath0[:, TD + 3]
          - jnp.broadcast_to(rif.reshape(NBLOCKS, 1, NBLK),
                             (NBLOCKS, K, NBLK)).reshape(E))
    e_s = jnp.concatenate([_rbf(_norm(evec), 0.0, 20.0, 16),
                           _pos_embed(rd, 16)], -1)
    e_v = _normalize(evec)
    esf, evf = _embed_edge_call(e_s, e_v, params["embed_edge"])

    gath = gath0
    for lp in params["layers"]:
        table = _layer_call(table, gath, esf, evf, _layer_weights(lp))
        if lp is not params["layers"][-1]:
            gath = _sc_gather(table, idx3d)

    s = table[:, :NS].reshape(b, l, NS)
    V = jnp.stack([table[:, NS + c * NV:NS + (c + 1) * NV]
                   for c in range(3)], axis=-1).reshape(b, l, NV, 3)
    return s, V
